# baseline (speedup 1.0000x reference)
import functools

import jax
import jax.numpy as jnp
from jax import lax
from jax.experimental import pallas as pl
from jax.experimental.pallas import tpu as pltpu
from jax.experimental.pallas import tpu_sc as plsc

_V = 1000000
_D = 64
_A = 4096
_S = 200
_SCALE = float(_D) ** 0.5
_K = 8192
_Q = _K // 4
_E = _K // 8
_BLKS = (_V + _K - 1) // _K
_RA_ROWS = _BLKS * _Q
_RB_ROWS = _BLKS * _E
_NCHUNK = (_V + 127) // 128
_CPW = (_NCHUNK + 31) // 32
_RC_ROWS = 32 * _CPW * 16


def _k1tc_body(t_ref, oa_ref, ob_ref):
    blk = t_ref[...]
    for q in range(4):
        tq = jnp.transpose(blk[:, q * _Q:(q + 1) * _Q], (1, 0)) * _SCALE
        oa_ref[:, 32 * q:32 * (q + 1)] = tq[:, 0:32]
        ob_ref[:, 32 * q:32 * q + 16] = tq[0:_E, 32:48]
        ob_ref[:, 32 * q + 16:32 * q + 32] = tq[_E:2 * _E, 32:48]


def _k1tc(tT):
    return pl.pallas_call(
        _k1tc_body,
        grid=(_BLKS,),
        in_specs=[pl.BlockSpec((_D, _K), lambda i: (0, i))],
        out_specs=[
            pl.BlockSpec((_Q, 128), lambda i: (i, 0)),
            pl.BlockSpec((_E, 128), lambda i: (i, 0)),
        ],
        out_shape=[
            jax.ShapeDtypeStruct((_RA_ROWS, 128), jnp.float32),
            jax.ShapeDtypeStruct((_RB_ROWS, 128), jnp.float32),
        ],
    )(tT)


@functools.lru_cache(maxsize=None)
def _make_k1sc():
    info = plsc.get_sparse_core_info()
    nc, ns = info.num_cores, info.num_subcores
    mesh = plsc.VectorSubcoreMesh(core_axis_name="c", subcore_axis_name="s")

    @functools.partial(
        pl.kernel,
        out_type=jax.ShapeDtypeStruct((_RC_ROWS, 128), jnp.float32),
        mesh=mesh,
        scratch_types=[
            pltpu.VMEM((2, 16, 128), jnp.float32),
            pltpu.VMEM((2, 16, 128), jnp.float32),
            pltpu.SemaphoreType.DMA,
            pltpu.SemaphoreType.DMA,
            pltpu.SemaphoreType.DMA,
            pltpu.SemaphoreType.DMA,
        ],
        compiler_params=pltpu.CompilerParams(
            use_tc_tiling_on_sc=True, needs_layout_passes=False
        ),
    )
    def k1sc(tT_hbm, tail_hbm, rc_hbm, slab, obuf, i0, i1, o0, o1):
        isem = (i0, i1)
        osem = (o0, o1)
        wid = lax.axis_index("s") * nc + lax.axis_index("c")
        c0 = wid * _CPW

        iota = lax.iota(jnp.int32, 16)

        def read(b, c):
            safe = pl.multiple_of(jnp.where(c < _NCHUNK, c, 0) * 128, 128)

            @pl.when(c == _NCHUNK - 1)
            def _tail():
                pltpu.async_copy(
                    tail_hbm.at[pl.ds(48, 16)], slab.at[b], isem[b]
                )

            @pl.when(c != _NCHUNK - 1)
            def _main():
                pltpu.async_copy(
                    tT_hbm.at[pl.ds(48, 16), pl.ds(safe, 128)],
                    slab.at[b],
                    isem[b],
                )

        def wait_read(b, c):
            pltpu.make_async_copy(
                tT_hbm.at[pl.ds(48, 16), pl.ds(0, 128)], slab.at[b], isem[b]
            ).wait()

        def transpose(b, c):
            @plsc.parallel_loop(0, 128, 1, unroll=8)
            def _dloop(d):
                m = (d + iota) & 127
                vec = plsc.load_gather(slab.at[b], [iota, m])
                plsc.store_scatter(
                    obuf.at[b],
                    [m >> 3, (m & 7) * 16 + iota],
                    vec * _SCALE,
                )

        def write(b, c):
            pltpu.async_copy(
                obuf.at[b], rc_hbm.at[pl.ds(c * 16, 16)], osem[b]
            )

        def wait_write(b, c):
            pltpu.make_async_copy(
                obuf.at[b], rc_hbm.at[pl.ds(c * 16, 16)], osem[b]
            ).wait()

        read(0, c0)
        read(1, c0 + 1)

        def pair_body(i, carry):
            for b in range(2):
                cl = 2 * i + b
                c = c0 + cl

                @pl.when(cl < _CPW)
                def _do():
                    wait_read(b, c)

                    @pl.when(cl >= 2)
                    def _reuse():
                        wait_write(b, c - 2)

                    transpose(b, c)
                    write(b, c)

                    @pl.when(cl + 2 < _CPW)
                    def _next():
                        read(b, c + 2)

            return carry

        lax.fori_loop(0, (_CPW + 1) // 2, pair_body, 0)

        wait_write(0, c0 + _CPW - 1)
        wait_write(1, c0 + _CPW - 2)

    return k1sc


@functools.lru_cache(maxsize=None)
def _make_k2():
    info = plsc.get_sparse_core_info()
    nc, ns = info.num_cores, info.num_subcores
    nw = nc * ns
    assert _A % 128 == 0 and _A // 128 == nw
    per_w = 128 * _S

    mesh = plsc.VectorSubcoreMesh(core_axis_name="c", subcore_axis_name="s")

    @functools.partial(
        pl.kernel,
        out_type=jax.ShapeDtypeStruct((_S, 8, _A // 128, 8, 128), jnp.float32),
        mesh=mesh,
        scratch_types=[
            pltpu.VMEM((per_w,), jnp.int32),
            pltpu.VMEM((2, 128), jnp.int32),
            pltpu.VMEM((2, 128), jnp.int32),
            pltpu.VMEM((2, 128), jnp.int32),
            pltpu.VMEM((2, 128, 32), jnp.float32),
            pltpu.VMEM((2, 128, 16), jnp.float32),
            pltpu.VMEM((2, 128, 16), jnp.float32),
            pltpu.VMEM((2, 8, 8, 128), jnp.float32),
            pltpu.SemaphoreType.DMA,
            pltpu.SemaphoreType.DMA,
            pltpu.SemaphoreType.DMA,
            pltpu.SemaphoreType.DMA,
        ],
        compiler_params=pltpu.CompilerParams(
            use_tc_tiling_on_sc=False, needs_layout_passes=False
        ),
    )
    def k2(idx_hbm, ra_hbm, rb_hbm, rc_hbm, out_hbm,
           idxv, pa, pb, pc, ga, gb, gc, o_v, s0, s1, w0, w1):
        gsem = (s0, s1)
        wsem = (w0, w1)
        wid = lax.axis_index("s") * nc + lax.axis_index("c")
        pltpu.sync_copy(idx_hbm.at[pl.ds(wid * per_w, per_w)], idxv)

        iota = lax.iota(jnp.int32, 16)
        iota_s = iota * _S

        def extract(b, s):
            for g in range(8):
                v = plsc.load_gather(idxv, [iota_s + (g * 16 * _S + s)])
                hi = v & ~(_K - 1)
                pa[b, pl.ds(g * 16, 16)] = (
                    hi | ((v & (_Q - 1)) << 2) | ((v >> 11) & 3)
                )
                pb[b, pl.ds(g * 16, 16)] = (
                    hi | ((v & (_E - 1)) << 3) | ((v >> 10) & 7)
                )
                pc[b, pl.ds(g * 16, 16)] = v

        def fire(b):
            pltpu.async_copy(ra_hbm.at[pa.at[b]], ga.at[b], gsem[b])
            pltpu.async_copy(rb_hbm.at[pb.at[b]], gb.at[b], gsem[b])
            pltpu.async_copy(rc_hbm.at[pc.at[b]], gc.at[b], gsem[b])

        def wait_gather(b):
            pltpu.make_async_copy(ra_hbm.at[pa.at[b]], ga.at[b], gsem[b]).wait()
            pltpu.make_async_copy(rb_hbm.at[pb.at[b]], gb.at[b], gsem[b]).wait()
            pltpu.make_async_copy(rc_hbm.at[pc.at[b]], gc.at[b], gsem[b]).wait()

        def shuffle(b):
            @plsc.parallel_loop(0, 32, 1, unroll=8)
            def _ja(j):
                t = (j + iota) & 31
                for g in range(8):
                    la = iota + g * 16
                    vec = plsc.load_gather(ga.at[b], [la, t])
                    plsc.store_scatter(o_v.at[b], [t >> 3, t & 7, la], vec)

            @plsc.parallel_loop(0, 16, 1, unroll=8)
            def _jb(j):
                t = (j + iota) & 15
                for g in range(8):
                    la = iota + g * 16
                    vec = plsc.load_gather(gb.at[b], [la, t])
                    plsc.store_scatter(o_v.at[b], [4 + (t >> 3), t & 7, la], vec)

            @plsc.parallel_loop(0, 16, 1, unroll=8)
            def _jc(j):
                t = (j + iota) & 15
                for g in range(8):
                    la = iota + g * 16
                    vec = plsc.load_gather(gc.at[b], [la, t])
                    plsc.store_scatter(o_v.at[b], [6 + (t >> 3), t & 7, la], vec)

        def write(b, s):
            pltpu.async_copy(o_v.at[b], out_hbm.at[s, :, wid], wsem[b])

        def wait_write(b, s):
            pltpu.make_async_copy(
                o_v.at[b], out_hbm.at[s, :, wid], wsem[b]
            ).wait()

        extract(0, 0)
        fire(0)
        extract(1, 1)
        fire(1)

        def pair_body(i, carry):
            for b in range(2):
                s = 2 * i + b
                wait_gather(b)

                @pl.when(s >= 2)
                def _reuse():
                    wait_write(b, s - 2)

                shuffle(b)
                write(b, s)

                @pl.when(s + 2 < _S)
                def _next():
                    extract(b, s + 2)
                    fire(b)

            return carry

        lax.fori_loop(0, _S // 2, pair_body, 0)

        wait_write(0, _S - 2)
        wait_write(1, _S - 1)

    return k2


def kernel(input, table):
    tT = table.T
    ra, rb = _k1tc(tT)
    n_tail = _V - 128 * (_NCHUNK - 1)
    tail = jnp.pad(
        lax.slice(tT, (0, 128 * (_NCHUNK - 1)), (_D, _V)),
        ((0, 0), (0, 128 - n_tail)),
    )
    rc = _make_k1sc()(tT, tail)
    ra2 = ra.reshape(4 * _RA_ROWS, 32)
    rb2 = rb.reshape(8 * _RB_ROWS, 16)
    rc2 = rc.reshape(8 * _RC_ROWS, 16)
    idxf = input.reshape(_A * _S).astype(jnp.int32)
    out5 = _make_k2()(idxf, ra2, rb2, rc2)
    return out5.transpose(2, 4, 0, 1, 3).reshape(_A, _S, _D)


# speedup vs baseline: 1.3170x; 1.3170x over previous
import functools

import jax
import jax.numpy as jnp
from jax import lax
from jax.experimental import pallas as pl
from jax.experimental.pallas import tpu as pltpu
from jax.experimental.pallas import tpu_sc as plsc

_V = 1000000
_D = 64
_A = 4096
_S = 200
_SCALE = float(_D) ** 0.5
_K1_COLS = 8192
_K1_HALF = _K1_COLS // 2
_K1_SHIFT = _K1_HALF.bit_length() - 1
_K1_BLKS = (_V + _K1_COLS - 1) // _K1_COLS
_R_ROWS = _K1_BLKS * _K1_HALF


def _k1_body(t_ref, o_ref):
    blk = t_ref[...]
    o_ref[:, 0:_D] = jnp.transpose(blk[:, :_K1_HALF], (1, 0)) * _SCALE
    o_ref[:, _D:128] = jnp.transpose(blk[:, _K1_HALF:], (1, 0)) * _SCALE


def _k1(tT):
    return pl.pallas_call(
        _k1_body,
        grid=(_K1_BLKS,),
        in_specs=[pl.BlockSpec((_D, _K1_COLS), lambda i: (0, i))],
        out_specs=pl.BlockSpec((_K1_HALF, 128), lambda i: (i, 0)),
        out_shape=jax.ShapeDtypeStruct((_R_ROWS, 128), jnp.float32),
    )(tT)


@functools.lru_cache(maxsize=None)
def _make_k2():
    info = plsc.get_sparse_core_info()
    nc, ns = info.num_cores, info.num_subcores
    nw = nc * ns
    assert _A % 128 == 0 and _A // 128 == nw
    per_w = 128 * _S

    mesh = plsc.VectorSubcoreMesh(core_axis_name="c", subcore_axis_name="s")

    @functools.partial(
        pl.kernel,
        out_type=jax.ShapeDtypeStruct((_S, 8, _A // 128, 8, 128), jnp.float32),
        mesh=mesh,
        scratch_types=[
            pltpu.VMEM((per_w,), jnp.int32),
            pltpu.VMEM((2, 128), jnp.int32),
            pltpu.VMEM((2, 128, _D), jnp.float32),
            pltpu.VMEM((2, 8, 8, 128), jnp.float32),
            pltpu.SemaphoreType.DMA,
            pltpu.SemaphoreType.DMA,
            pltpu.SemaphoreType.DMA,
            pltpu.SemaphoreType.DMA,
        ],
        compiler_params=pltpu.CompilerParams(
            use_tc_tiling_on_sc=False, needs_layout_passes=False
        ),
    )
    def k2(idx_hbm, r_hbm, out_hbm, idxv, pbuf, g_v, o_v, s0, s1, w0, w1):
        gsem = (s0, s1)
        wsem = (w0, w1)
        wid = lax.axis_index("s") * nc + lax.axis_index("c")
        pltpu.sync_copy(idx_hbm.at[pl.ds(wid * per_w, per_w)], idxv)

        iota = lax.iota(jnp.int32, 16)
        iota_s = iota * _S

        def extract(b, s):
            for g in range(8):
                v = plsc.load_gather(idxv, [iota_s + (g * 16 * _S + s)])
                phys = (
                    (v & ~(_K1_COLS - 1))
                    | ((v & (_K1_HALF - 1)) << 1)
                    | ((v >> _K1_SHIFT) & 1)
                )
                pbuf[b, pl.ds(g * 16, 16)] = phys

        def fire(b):
            pltpu.async_copy(r_hbm.at[pbuf.at[b]], g_v.at[b], gsem[b])

        def wait_gather(b):
            pltpu.make_async_copy(
                r_hbm.at[pbuf.at[b]], g_v.at[b], gsem[b]
            ).wait()

        def shuffle(b):
            @plsc.parallel_loop(0, _D, 1, unroll=16)
            def _jloop(j):
                t = (j + iota) & (_D - 1)
                tj = t >> 3
                sj = t & 7
                for g in range(8):
                    vec = plsc.load_gather(g_v.at[b], [iota + g * 16, t])
                    plsc.store_scatter(o_v.at[b], [tj, sj, iota + g * 16], vec)

        def write(b, s):
            pltpu.async_copy(o_v.at[b], out_hbm.at[s, :, wid], wsem[b])

        def wait_write(b, s):
            pltpu.make_async_copy(
                o_v.at[b], out_hbm.at[s, :, wid], wsem[b]
            ).wait()

        extract(0, 0)
        fire(0)
        extract(1, 1)
        fire(1)

        def pair_body(i, carry):
            for b in range(2):
                s = 2 * i + b
                wait_gather(b)

                @pl.when(s >= 2)
                def _reuse():
                    wait_write(b, s - 2)

                shuffle(b)
                write(b, s)

                @pl.when(s + 2 < _S)
                def _next():
                    extract(b, s + 2)
                    fire(b)

            return carry

        lax.fori_loop(0, _S // 2, pair_body, 0)

        wait_write(0, _S - 2)
        wait_write(1, _S - 1)

    return k2


def kernel(input, table):
    tT = table.T
    r = _k1(tT)
    r2 = r.reshape(2 * _R_ROWS, _D)
    idxf = input.reshape(_A * _S).astype(jnp.int32)
    out5 = _make_k2()(idxf, r2)
    return out5.transpose(2, 4, 0, 1, 3).reshape(_A, _S, _D)


# speedup vs baseline: 1.4142x; 1.0737x over previous
import functools

import jax
import jax.numpy as jnp
from jax import lax
from jax.experimental import pallas as pl
from jax.experimental.pallas import tpu as pltpu
from jax.experimental.pallas import tpu_sc as plsc

_V = 1000000
_D = 64
_A = 4096
_S = 200
_SCALE = float(_D) ** 0.5
_K1_COLS = 16384
_K1_HALF = _K1_COLS // 2
_K1_SHIFT = _K1_HALF.bit_length() - 1
_K1_BLKS = (_V + _K1_COLS - 1) // _K1_COLS
_R_ROWS = _K1_BLKS * _K1_HALF


def _k1_body(t_ref, o_ref):
    blk = t_ref[...]
    o_ref[:, 0:_D] = jnp.transpose(blk[:, :_K1_HALF], (1, 0)) * _SCALE
    o_ref[:, _D:128] = jnp.transpose(blk[:, _K1_HALF:], (1, 0)) * _SCALE


def _k1(tT):
    return pl.pallas_call(
        _k1_body,
        grid=(_K1_BLKS,),
        in_specs=[pl.BlockSpec((_D, _K1_COLS), lambda i: (0, i))],
        out_specs=pl.BlockSpec((_K1_HALF, 128), lambda i: (i, 0)),
        out_shape=jax.ShapeDtypeStruct((_R_ROWS, 128), jnp.float32),
    )(tT)


@functools.lru_cache(maxsize=None)
def _make_k2():
    info = plsc.get_sparse_core_info()
    nc, ns = info.num_cores, info.num_subcores
    nw = nc * ns
    assert _A % 128 == 0 and _A // 128 == nw
    per_w = 128 * _S

    mesh = plsc.VectorSubcoreMesh(core_axis_name="c", subcore_axis_name="s")

    @functools.partial(
        pl.kernel,
        out_type=jax.ShapeDtypeStruct((_S, 8, _A // 128, 8, 128), jnp.float32),
        mesh=mesh,
        scratch_types=[
            pltpu.VMEM((per_w,), jnp.int32),
            pltpu.VMEM((2, 128), jnp.int32),
            pltpu.VMEM((2, 128, _D), jnp.float32),
            pltpu.VMEM((2, 8, 8, 128), jnp.float32),
            pltpu.SemaphoreType.DMA,
            pltpu.SemaphoreType.DMA,
            pltpu.SemaphoreType.DMA,
            pltpu.SemaphoreType.DMA,
        ],
        compiler_params=pltpu.CompilerParams(
            use_tc_tiling_on_sc=False, needs_layout_passes=False
        ),
    )
    def k2(idx_hbm, r_hbm, out_hbm, idxv, pbuf, g_v, o_v, s0, s1, w0, w1):
        gsem = (s0, s1)
        wsem = (w0, w1)
        wid = lax.axis_index("s") * nc + lax.axis_index("c")
        pltpu.sync_copy(idx_hbm.at[pl.ds(wid * per_w, per_w)], idxv)

        iota = lax.iota(jnp.int32, 16)
        iota_s = iota * _S

        def extract(b, s):
            for g in range(8):
                v = plsc.load_gather(idxv, [iota_s + (g * 16 * _S + s)])
                phys = (
                    (v & ~(_K1_COLS - 1))
                    | ((v & (_K1_HALF - 1)) << 1)
                    | ((v >> _K1_SHIFT) & 1)
                )
                pbuf[b, pl.ds(g * 16, 16)] = phys

        def fire(b):
            pltpu.async_copy(r_hbm.at[pbuf.at[b]], g_v.at[b], gsem[b])

        def wait_gather(b):
            pltpu.make_async_copy(
                r_hbm.at[pbuf.at[b]], g_v.at[b], gsem[b]
            ).wait()

        def shuffle(b):
            @plsc.parallel_loop(0, _D, 1, unroll=16)
            def _jloop(j):
                t = (j + iota) & (_D - 1)
                tj = t >> 3
                sj = t & 7
                for g in range(8):
                    vec = plsc.load_gather(g_v.at[b], [iota + g * 16, t])
                    plsc.store_scatter(o_v.at[b], [tj, sj, iota + g * 16], vec)

        def write(b, s):
            pltpu.async_copy(o_v.at[b], out_hbm.at[s, :, wid], wsem[b])

        def wait_write(b, s):
            pltpu.make_async_copy(
                o_v.at[b], out_hbm.at[s, :, wid], wsem[b]
            ).wait()

        extract(0, 0)
        fire(0)
        extract(1, 1)
        fire(1)

        def pair_body(i, carry):
            for b in range(2):
                s = 2 * i + b
                wait_gather(b)

                @pl.when(s >= 2)
                def _reuse():
                    wait_write(b, s - 2)

                shuffle(b)
                write(b, s)

                @pl.when(s + 2 < _S)
                def _next():
                    extract(b, s + 2)
                    fire(b)

            return carry

        lax.fori_loop(0, _S // 2, pair_body, 0)

        wait_write(0, _S - 2)
        wait_write(1, _S - 1)

    return k2


def kernel(input, table):
    tT = table.T
    r = _k1(tT)
    r2 = r.reshape(2 * _R_ROWS, _D)
    idxf = input.reshape(_A * _S).astype(jnp.int32)
    out5 = _make_k2()(idxf, r2)
    return out5.transpose(2, 4, 0, 1, 3).reshape(_A, _S, _D)


# speedup vs baseline: 1.4644x; 1.0355x over previous
import functools

import jax
import jax.numpy as jnp
from jax import lax
from jax.experimental import pallas as pl
from jax.experimental.pallas import tpu as pltpu
from jax.experimental.pallas import tpu_sc as plsc

_V = 1000000
_D = 64
_A = 4096
_S = 200
_SCALE = float(_D) ** 0.5
_K1_COLS = 32768
_K1_HALF = _K1_COLS // 2
_K1_SHIFT = _K1_HALF.bit_length() - 1
_K1_BLKS = (_V + _K1_COLS - 1) // _K1_COLS
_R_ROWS = _K1_BLKS * _K1_HALF


def _k1_body(t_ref, o_ref):
    blk = t_ref[...]
    o_ref[:, 0:_D] = jnp.transpose(blk[:, :_K1_HALF], (1, 0)) * _SCALE
    o_ref[:, _D:128] = jnp.transpose(blk[:, _K1_HALF:], (1, 0)) * _SCALE


def _k1(tT):
    return pl.pallas_call(
        _k1_body,
        grid=(_K1_BLKS,),
        in_specs=[pl.BlockSpec((_D, _K1_COLS), lambda i: (0, i))],
        out_specs=pl.BlockSpec((_K1_HALF, 128), lambda i: (i, 0)),
        out_shape=jax.ShapeDtypeStruct((_R_ROWS, 128), jnp.float32),
    )(tT)


@functools.lru_cache(maxsize=None)
def _make_k2():
    info = plsc.get_sparse_core_info()
    nc, ns = info.num_cores, info.num_subcores
    nw = nc * ns
    assert _A % 128 == 0 and _A // 128 == nw
    per_w = 128 * _S

    mesh = plsc.VectorSubcoreMesh(core_axis_name="c", subcore_axis_name="s")

    @functools.partial(
        pl.kernel,
        out_type=jax.ShapeDtypeStruct((_S, 8, _A // 128, 8, 128), jnp.float32),
        mesh=mesh,
        scratch_types=[
            pltpu.VMEM((per_w,), jnp.int32),
            pltpu.VMEM((2, 128), jnp.int32),
            pltpu.VMEM((2, 128, _D), jnp.float32),
            pltpu.VMEM((2, 8, 8, 128), jnp.float32),
            pltpu.SemaphoreType.DMA,
            pltpu.SemaphoreType.DMA,
            pltpu.SemaphoreType.DMA,
            pltpu.SemaphoreType.DMA,
        ],
        compiler_params=pltpu.CompilerParams(
            use_tc_tiling_on_sc=False, needs_layout_passes=False
        ),
    )
    def k2(idx_hbm, r_hbm, out_hbm, idxv, pbuf, g_v, o_v, s0, s1, w0, w1):
        gsem = (s0, s1)
        wsem = (w0, w1)
        wid = lax.axis_index("s") * nc + lax.axis_index("c")
        pltpu.sync_copy(idx_hbm.at[pl.ds(wid * per_w, per_w)], idxv)

        iota = lax.iota(jnp.int32, 16)
        iota_s = iota * _S

        def extract(b, s):
            for g in range(8):
                v = plsc.load_gather(idxv, [iota_s + (g * 16 * _S + s)])
                phys = (
                    (v & ~(_K1_COLS - 1))
                    | ((v & (_K1_HALF - 1)) << 1)
                    | ((v >> _K1_SHIFT) & 1)
                )
                pbuf[b, pl.ds(g * 16, 16)] = phys

        def fire(b):
            pltpu.async_copy(r_hbm.at[pbuf.at[b]], g_v.at[b], gsem[b])

        def wait_gather(b):
            pltpu.make_async_copy(
                r_hbm.at[pbuf.at[b]], g_v.at[b], gsem[b]
            ).wait()

        def shuffle(b):
            @plsc.parallel_loop(0, _D, 1, unroll=16)
            def _jloop(j):
                t = (j + iota) & (_D - 1)
                tj = t >> 3
                sj = t & 7
                for g in range(8):
                    vec = plsc.load_gather(g_v.at[b], [iota + g * 16, t])
                    plsc.store_scatter(o_v.at[b], [tj, sj, iota + g * 16], vec)

        def write(b, s):
            pltpu.async_copy(o_v.at[b], out_hbm.at[s, :, wid], wsem[b])

        def wait_write(b, s):
            pltpu.make_async_copy(
                o_v.at[b], out_hbm.at[s, :, wid], wsem[b]
            ).wait()

        extract(0, 0)
        fire(0)
        extract(1, 1)
        fire(1)

        def pair_body(i, carry):
            for b in range(2):
                s = 2 * i + b
                wait_gather(b)

                @pl.when(s >= 2)
                def _reuse():
                    wait_write(b, s - 2)

                shuffle(b)
                write(b, s)

                @pl.when(s + 2 < _S)
                def _next():
                    extract(b, s + 2)
                    fire(b)

            return carry

        lax.fori_loop(0, _S // 2, pair_body, 0)

        wait_write(0, _S - 2)
        wait_write(1, _S - 1)

    return k2


def kernel(input, table):
    tT = table.T
    r = _k1(tT)
    r2 = r.reshape(2 * _R_ROWS, _D)
    idxf = input.reshape(_A * _S).astype(jnp.int32)
    out5 = _make_k2()(idxf, r2)
    return out5.transpose(2, 4, 0, 1, 3).reshape(_A, _S, _D)
